# Initial kernel scaffold; baseline (speedup 1.0000x reference)
import jax
import jax.numpy as jnp
from jax.experimental import pallas as pl


def kernel(x, w1, g1, b1, w2, g2, b2):
    raise NotImplementedError("write your pallas kernel here")


# speedup vs baseline: 1.4514x; 1.4514x over previous
import functools

import jax
import jax.numpy as jnp
from jax.experimental import pallas as pl
from jax.experimental.pallas import tpu as pltpu

_EPS = 1e-5

_H = 56
_W = 56
_C = 128
_WP = _W + 2
_ROWS = _H * _WP
_FLAT = (_H + 2) * _WP
_FLATP = 3368
_OFF = _WP + 1
_TAPS = tuple(dh * _WP + dw for dh in range(3) for dw in range(3))


def _conv_from_padded(xp, w_ref):
    col = jnp.concatenate(
        [jax.lax.slice(xp, (t, 0), (t + _ROWS, _C)) for t in _TAPS],
        axis=1)
    return jnp.dot(col, w_ref[...], preferred_element_type=jnp.float32)


def _conv1_kernel(xp_ref, w_ref, mask_ref, y_ref, stats_ref):
    y = _conv_from_padded(xp_ref[...], w_ref)
    ym = y * mask_ref[...]
    stats_ref[0:1, :] = jnp.sum(ym, axis=0, keepdims=True)
    stats_ref[1:2, :] = jnp.sum(ym * ym, axis=0, keepdims=True)
    y_ref[...] = ym.astype(jnp.bfloat16)


def _bn_conv2_kernel(y1_ref, scale_ref, shift_ref, w_ref, mask_ref,
                     y_ref, stats_ref):
    a = jnp.maximum(y1_ref[...].astype(jnp.float32) * scale_ref[...]
                    + shift_ref[...], 0.0)
    a = (a * mask_ref[...]).astype(jnp.bfloat16)
    ap = jnp.concatenate(
        [jnp.zeros((_OFF, _C), jnp.bfloat16), a,
         jnp.zeros((_FLATP - _OFF - _ROWS, _C), jnp.bfloat16)], axis=0)
    y = _conv_from_padded(ap, w_ref)
    ym = y * mask_ref[...]
    stats_ref[0:1, :] = jnp.sum(ym, axis=0, keepdims=True)
    stats_ref[1:2, :] = jnp.sum(ym * ym, axis=0, keepdims=True)
    y_ref[...] = ym.astype(jnp.bfloat16)


def _bn_out_kernel(y2_ref, scale_ref, shift_ref, o_ref):
    a = jnp.maximum(y2_ref[...].astype(jnp.float32) * scale_ref[...]
                    + shift_ref[...], 0.0)
    compact = jnp.concatenate(
        [jax.lax.slice(a, (h * _WP, 0), (h * _WP + _W, _C))
         for h in range(_H)], axis=0)
    o_ref[...] = compact


def _fold_bn(stats, gamma, beta, count):
    ssum = jnp.sum(stats[:, 0, :], axis=0)
    ssq = jnp.sum(stats[:, 1, :], axis=0)
    mean = ssum / count
    var = jnp.maximum(ssq / count - mean * mean, 0.0)
    inv = jax.lax.rsqrt(var + _EPS)
    scale = gamma.astype(jnp.float32) * inv
    shift = beta.astype(jnp.float32) - mean * scale
    return scale.reshape(1, _C), shift.reshape(1, _C)


def _wmat(w_oihw):
    return jnp.transpose(w_oihw, (2, 3, 1, 0)).reshape(9 * _C, _C).astype(
        jnp.bfloat16)


@jax.jit
def kernel(x, w1, g1, b1, w2, g2, b2):
    n = x.shape[0]
    count = float(n * _H * _W)

    x_nhwc = jnp.transpose(x, (0, 2, 3, 1)).astype(jnp.bfloat16)
    xp = jnp.pad(x_nhwc, ((0, 0), (1, 1), (1, 1), (0, 0)))
    xp = jnp.pad(xp.reshape(n, _FLAT, _C), ((0, 0), (0, _FLATP - _FLAT),
                                            (0, 0)))

    mask = (jnp.arange(_ROWS) % _WP < _W).astype(jnp.float32)
    mask = jnp.broadcast_to(mask[:, None], (_ROWS, _C))

    y1, stats1 = pl.pallas_call(
        _conv1_kernel,
        grid=(n,),
        in_specs=[
            pl.BlockSpec((None, _FLATP, _C), lambda i: (i, 0, 0)),
            pl.BlockSpec((9 * _C, _C), lambda i: (0, 0)),
            pl.BlockSpec((_ROWS, _C), lambda i: (0, 0)),
        ],
        out_shape=(
            jax.ShapeDtypeStruct((n, _ROWS, _C), jnp.bfloat16),
            jax.ShapeDtypeStruct((n, 2, _C), jnp.float32),
        ),
        out_specs=(
            pl.BlockSpec((None, _ROWS, _C), lambda i: (i, 0, 0)),
            pl.BlockSpec((None, 2, _C), lambda i: (i, 0, 0)),
        ),
        compiler_params=pltpu.CompilerParams(
            dimension_semantics=("parallel",)),
    )(xp, _wmat(w1), mask)

    scale1, shift1 = _fold_bn(stats1, g1, b1, count)

    y2, stats2 = pl.pallas_call(
        _bn_conv2_kernel,
        grid=(n,),
        in_specs=[
            pl.BlockSpec((None, _ROWS, _C), lambda i: (i, 0, 0)),
            pl.BlockSpec((1, _C), lambda i: (0, 0)),
            pl.BlockSpec((1, _C), lambda i: (0, 0)),
            pl.BlockSpec((9 * _C, _C), lambda i: (0, 0)),
            pl.BlockSpec((_ROWS, _C), lambda i: (0, 0)),
        ],
        out_shape=(
            jax.ShapeDtypeStruct((n, _ROWS, _C), jnp.bfloat16),
            jax.ShapeDtypeStruct((n, 2, _C), jnp.float32),
        ),
        out_specs=(
            pl.BlockSpec((None, _ROWS, _C), lambda i: (i, 0, 0)),
            pl.BlockSpec((None, 2, _C), lambda i: (i, 0, 0)),
        ),
        compiler_params=pltpu.CompilerParams(
            dimension_semantics=("parallel",)),
    )(y1, scale1, shift1, _wmat(w2), mask)

    scale2, shift2 = _fold_bn(stats2, g2, b2, count)

    out = pl.pallas_call(
        _bn_out_kernel,
        grid=(n,),
        in_specs=[
            pl.BlockSpec((None, _ROWS, _C), lambda i: (i, 0, 0)),
            pl.BlockSpec((1, _C), lambda i: (0, 0)),
            pl.BlockSpec((1, _C), lambda i: (0, 0)),
        ],
        out_shape=jax.ShapeDtypeStruct((n, _H * _W, _C), jnp.float32),
        out_specs=pl.BlockSpec((None, _H * _W, _C), lambda i: (i, 0, 0)),
        compiler_params=pltpu.CompilerParams(
            dimension_semantics=("parallel",)),
    )(y2, scale2, shift2)

    return jnp.transpose(out.reshape(n, _H, _W, _C), (0, 3, 1, 2))


# speedup vs baseline: 1.4717x; 1.0140x over previous
import functools

import jax
import jax.numpy as jnp
from jax.experimental import pallas as pl
from jax.experimental.pallas import tpu as pltpu

_EPS = 1e-5

_H = 56
_W = 56
_C = 128
_WP = _W + 2
_ROWS = _H * _WP
_FLAT = (_H + 2) * _WP
_FLATP = 3368
_OFF = _WP + 1
_TAPS = tuple(dh * _WP + dw for dh in range(3) for dw in range(3))


def _conv_from_padded(xp, w_ref):
    col = jnp.concatenate(
        [jax.lax.slice(xp, (t, 0), (t + _ROWS, _C)) for t in _TAPS],
        axis=1)
    return jnp.dot(col, w_ref[...], preferred_element_type=jnp.float32)


def _conv1_kernel(x_ref, w_ref, mask_ref, y_ref, stats_ref, xp_ref):
    xt = jnp.transpose(x_ref[...].astype(jnp.bfloat16), (1, 0))
    xp_ref[...] = jnp.zeros((_FLATP, _C), jnp.bfloat16)
    for h in range(_H):
        xp_ref[_OFF + h * _WP:_OFF + h * _WP + _W, :] = (
            xt[h * _W:(h + 1) * _W, :])
    y = _conv_from_padded(xp_ref[...], w_ref)
    ym = y * mask_ref[...]
    stats_ref[0:1, :] = jnp.sum(ym, axis=0, keepdims=True)
    stats_ref[1:2, :] = jnp.sum(ym * ym, axis=0, keepdims=True)
    y_ref[...] = ym.astype(jnp.bfloat16)


def _bn_conv2_kernel(y1_ref, scale_ref, shift_ref, w_ref, mask_ref,
                     y_ref, stats_ref):
    a = jnp.maximum(y1_ref[...].astype(jnp.float32) * scale_ref[...]
                    + shift_ref[...], 0.0)
    a = (a * mask_ref[...]).astype(jnp.bfloat16)
    ap = jnp.concatenate(
        [jnp.zeros((_OFF, _C), jnp.bfloat16), a,
         jnp.zeros((_FLATP - _OFF - _ROWS, _C), jnp.bfloat16)], axis=0)
    y = _conv_from_padded(ap, w_ref)
    ym = y * mask_ref[...]
    stats_ref[0:1, :] = jnp.sum(ym, axis=0, keepdims=True)
    stats_ref[1:2, :] = jnp.sum(ym * ym, axis=0, keepdims=True)
    y_ref[...] = ym.astype(jnp.bfloat16)


def _bn_out_kernel(y2_ref, scale_ref, shift_ref, o_ref):
    a = jnp.maximum(y2_ref[...].astype(jnp.float32) * scale_ref[...]
                    + shift_ref[...], 0.0)
    compact = jnp.concatenate(
        [jax.lax.slice(a, (h * _WP, 0), (h * _WP + _W, _C))
         for h in range(_H)], axis=0)
    o_ref[...] = jnp.transpose(compact, (1, 0))


def _fold_bn(stats, gamma, beta, count):
    ssum = jnp.sum(stats[:, 0, :], axis=0)
    ssq = jnp.sum(stats[:, 1, :], axis=0)
    mean = ssum / count
    var = jnp.maximum(ssq / count - mean * mean, 0.0)
    inv = jax.lax.rsqrt(var + _EPS)
    scale = gamma.astype(jnp.float32) * inv
    shift = beta.astype(jnp.float32) - mean * scale
    return scale.reshape(1, _C), shift.reshape(1, _C)


def _wmat(w_oihw):
    return jnp.transpose(w_oihw, (2, 3, 1, 0)).reshape(9 * _C, _C).astype(
        jnp.bfloat16)


@jax.jit
def kernel(x, w1, g1, b1, w2, g2, b2):
    n = x.shape[0]
    count = float(n * _H * _W)

    xf = x.reshape(n, _C, _H * _W)

    mask = (jnp.arange(_ROWS) % _WP < _W).astype(jnp.float32)
    mask = jnp.broadcast_to(mask[:, None], (_ROWS, _C))

    y1, stats1 = pl.pallas_call(
        _conv1_kernel,
        grid=(n,),
        in_specs=[
            pl.BlockSpec((None, _C, _H * _W), lambda i: (i, 0, 0)),
            pl.BlockSpec((9 * _C, _C), lambda i: (0, 0)),
            pl.BlockSpec((_ROWS, _C), lambda i: (0, 0)),
        ],
        out_shape=(
            jax.ShapeDtypeStruct((n, _ROWS, _C), jnp.bfloat16),
            jax.ShapeDtypeStruct((n, 2, _C), jnp.float32),
        ),
        out_specs=(
            pl.BlockSpec((None, _ROWS, _C), lambda i: (i, 0, 0)),
            pl.BlockSpec((None, 2, _C), lambda i: (i, 0, 0)),
        ),
        scratch_shapes=[pltpu.VMEM((_FLATP, _C), jnp.bfloat16)],
        compiler_params=pltpu.CompilerParams(
            dimension_semantics=("parallel",)),
    )(xf, _wmat(w1), mask)

    scale1, shift1 = _fold_bn(stats1, g1, b1, count)

    y2, stats2 = pl.pallas_call(
        _bn_conv2_kernel,
        grid=(n,),
        in_specs=[
            pl.BlockSpec((None, _ROWS, _C), lambda i: (i, 0, 0)),
            pl.BlockSpec((1, _C), lambda i: (0, 0)),
            pl.BlockSpec((1, _C), lambda i: (0, 0)),
            pl.BlockSpec((9 * _C, _C), lambda i: (0, 0)),
            pl.BlockSpec((_ROWS, _C), lambda i: (0, 0)),
        ],
        out_shape=(
            jax.ShapeDtypeStruct((n, _ROWS, _C), jnp.bfloat16),
            jax.ShapeDtypeStruct((n, 2, _C), jnp.float32),
        ),
        out_specs=(
            pl.BlockSpec((None, _ROWS, _C), lambda i: (i, 0, 0)),
            pl.BlockSpec((None, 2, _C), lambda i: (i, 0, 0)),
        ),
        compiler_params=pltpu.CompilerParams(
            dimension_semantics=("parallel",)),
    )(y1, scale1, shift1, _wmat(w2), mask)

    scale2, shift2 = _fold_bn(stats2, g2, b2, count)

    out = pl.pallas_call(
        _bn_out_kernel,
        grid=(n,),
        in_specs=[
            pl.BlockSpec((None, _ROWS, _C), lambda i: (i, 0, 0)),
            pl.BlockSpec((1, _C), lambda i: (0, 0)),
            pl.BlockSpec((1, _C), lambda i: (0, 0)),
        ],
        out_shape=jax.ShapeDtypeStruct((n, _C, _H * _W), jnp.float32),
        out_specs=pl.BlockSpec((None, _C, _H * _W), lambda i: (i, 0, 0)),
        compiler_params=pltpu.CompilerParams(
            dimension_semantics=("parallel",)),
    )(y2, scale2, shift2)

    return out.reshape(n, _C, _H, _W)


# speedup vs baseline: 1.5135x; 1.0284x over previous
import jax
import jax.numpy as jnp
from jax.experimental import pallas as pl
from jax.experimental.pallas import tpu as pltpu

_EPS = 1e-5

_H = 56
_W = 56
_C = 128
_WP = _W + 2
_ROWS = _H * _WP
_FLAT = (_H + 2) * _WP
_FLATP = 3368
_OFF = _WP + 1
_TAPS = tuple(dh * _WP + dw for dh in range(3) for dw in range(3))
_BS = 2
_BSO = 4


def _conv_from_padded(xp, w_ref):
    col = jnp.concatenate(
        [jax.lax.slice(xp, (t, 0), (t + _ROWS, _C)) for t in _TAPS],
        axis=1)
    return jnp.dot(col, w_ref[...], preferred_element_type=jnp.float32)


def _fold_bn(stats, g, b, count):
    ssum = jnp.sum(stats[:, 0:1, :], axis=0)
    ssq = jnp.sum(stats[:, 1:2, :], axis=0)
    mean = ssum / count
    var = jnp.maximum(ssq / count - mean * mean, 0.0)
    inv = jax.lax.rsqrt(var + _EPS)
    scale = g * inv
    shift = b - mean * scale
    return scale, shift


def _conv1_kernel(x_ref, w_ref, mask_ref, y_ref, stats_ref, xp_ref):
    for s in range(_BS):
        xt = jnp.transpose(x_ref[s].astype(jnp.bfloat16), (1, 0))
        xp_ref[...] = jnp.zeros((_FLATP, _C), jnp.bfloat16)
        for h in range(_H):
            xp_ref[_OFF + h * _WP:_OFF + h * _WP + _W, :] = (
                xt[h * _W:(h + 1) * _W, :])
        y = _conv_from_padded(xp_ref[...], w_ref)
        ym = y * mask_ref[...]
        stats_ref[s, 0:1, :] = jnp.sum(ym, axis=0, keepdims=True)
        stats_ref[s, 1:2, :] = jnp.sum(ym * ym, axis=0, keepdims=True)
        y_ref[s] = ym.astype(jnp.bfloat16)


def _bn_conv2_kernel(y1_ref, stats1_ref, g_ref, b_ref, w_ref, mask_ref,
                     y_ref, stats_ref):
    n = stats1_ref.shape[0]
    scale, shift = _fold_bn(stats1_ref[...], g_ref[...], b_ref[...],
                            float(n * _H * _W))
    for s in range(_BS):
        a = jnp.maximum(y1_ref[s].astype(jnp.float32) * scale + shift, 0.0)
        a = (a * mask_ref[...]).astype(jnp.bfloat16)
        ap = jnp.concatenate(
            [jnp.zeros((_OFF, _C), jnp.bfloat16), a,
             jnp.zeros((_FLATP - _OFF - _ROWS, _C), jnp.bfloat16)], axis=0)
        y = _conv_from_padded(ap, w_ref)
        ym = y * mask_ref[...]
        stats_ref[s, 0:1, :] = jnp.sum(ym, axis=0, keepdims=True)
        stats_ref[s, 1:2, :] = jnp.sum(ym * ym, axis=0, keepdims=True)
        y_ref[s] = ym.astype(jnp.bfloat16)


def _bn_out_kernel(y2_ref, stats2_ref, g_ref, b_ref, o_ref):
    n = stats2_ref.shape[0]
    scale, shift = _fold_bn(stats2_ref[...], g_ref[...], b_ref[...],
                            float(n * _H * _W))
    for s in range(_BSO):
        a = jnp.maximum(y2_ref[s].astype(jnp.float32) * scale + shift, 0.0)
        compact = jnp.concatenate(
            [jax.lax.slice(a, (h * _WP, 0), (h * _WP + _W, _C))
             for h in range(_H)], axis=0)
        o_ref[s] = jnp.transpose(compact, (1, 0))


def _wmat(w_oihw):
    return jnp.transpose(w_oihw, (2, 3, 1, 0)).reshape(9 * _C, _C).astype(
        jnp.bfloat16)


@jax.jit
def kernel(x, w1, g1, b1, w2, g2, b2):
    n = x.shape[0]

    xf = x.reshape(n, _C, _H * _W)

    mask = (jnp.arange(_ROWS) % _WP < _W).astype(jnp.float32)
    mask = jnp.broadcast_to(mask[:, None], (_ROWS, _C))

    y1, stats1 = pl.pallas_call(
        _conv1_kernel,
        grid=(n // _BS,),
        in_specs=[
            pl.BlockSpec((_BS, _C, _H * _W), lambda i: (i, 0, 0)),
            pl.BlockSpec((9 * _C, _C), lambda i: (0, 0)),
            pl.BlockSpec((_ROWS, _C), lambda i: (0, 0)),
        ],
        out_shape=(
            jax.ShapeDtypeStruct((n, _ROWS, _C), jnp.bfloat16),
            jax.ShapeDtypeStruct((n, 2, _C), jnp.float32),
        ),
        out_specs=(
            pl.BlockSpec((_BS, _ROWS, _C), lambda i: (i, 0, 0)),
            pl.BlockSpec((_BS, 2, _C), lambda i: (i, 0, 0)),
        ),
        scratch_shapes=[pltpu.VMEM((_FLATP, _C), jnp.bfloat16)],
        compiler_params=pltpu.CompilerParams(
            dimension_semantics=("parallel",)),
    )(xf, _wmat(w1), mask)

    y2, stats2 = pl.pallas_call(
        _bn_conv2_kernel,
        grid=(n // _BS,),
        in_specs=[
            pl.BlockSpec((_BS, _ROWS, _C), lambda i: (i, 0, 0)),
            pl.BlockSpec((n, 2, _C), lambda i: (0, 0, 0)),
            pl.BlockSpec((1, _C), lambda i: (0, 0)),
            pl.BlockSpec((1, _C), lambda i: (0, 0)),
            pl.BlockSpec((9 * _C, _C), lambda i: (0, 0)),
            pl.BlockSpec((_ROWS, _C), lambda i: (0, 0)),
        ],
        out_shape=(
            jax.ShapeDtypeStruct((n, _ROWS, _C), jnp.bfloat16),
            jax.ShapeDtypeStruct((n, 2, _C), jnp.float32),
        ),
        out_specs=(
            pl.BlockSpec((_BS, _ROWS, _C), lambda i: (i, 0, 0)),
            pl.BlockSpec((_BS, 2, _C), lambda i: (i, 0, 0)),
        ),
        compiler_params=pltpu.CompilerParams(
            dimension_semantics=("parallel",)),
    )(y1, stats1, g1.reshape(1, _C).astype(jnp.float32),
      b1.reshape(1, _C).astype(jnp.float32), _wmat(w2), mask)

    out = pl.pallas_call(
        _bn_out_kernel,
        grid=(n // _BSO,),
        in_specs=[
            pl.BlockSpec((_BSO, _ROWS, _C), lambda i: (i, 0, 0)),
            pl.BlockSpec((n, 2, _C), lambda i: (0, 0, 0)),
            pl.BlockSpec((1, _C), lambda i: (0, 0)),
            pl.BlockSpec((1, _C), lambda i: (0, 0)),
        ],
        out_shape=jax.ShapeDtypeStruct((n, _C, _H * _W), jnp.float32),
        out_specs=pl.BlockSpec((_BSO, _C, _H * _W), lambda i: (i, 0, 0)),
        compiler_params=pltpu.CompilerParams(
            dimension_semantics=("parallel",)),
    )(y2, stats2, g2.reshape(1, _C).astype(jnp.float32),
      b2.reshape(1, _C).astype(jnp.float32))

    return out.reshape(n, _C, _H, _W)
